# baseline (speedup 1.0000x reference)
import functools

import jax
import jax.numpy as jnp
from jax import lax
from jax.experimental import pallas as pl
from jax.experimental.pallas import tpu as pltpu
from jax.experimental.pallas import tpu_sc as plsc

BATCH = 16384
N_SPACE = 8
LANES = 16

_info = plsc.get_sparse_core_info()
_NC, _NS = _info.num_cores, _info.num_subcores
_NW = _NC * _NS
_B_PER_W = BATCH // _NW
_VREGS = _B_PER_W // LANES

_mesh = plsc.VectorSubcoreMesh(core_axis_name="c", subcore_axis_name="s")


@functools.partial(
    pl.kernel,
    mesh=_mesh,
    out_type=jax.ShapeDtypeStruct((BATCH,), jnp.float32),
    compiler_params=pltpu.CompilerParams(
        needs_layout_passes=False,
        disable_bounds_checks=True,
        disable_semaphore_checks=True,
        skip_device_barrier=True,
    ),
    scratch_types=[
        pltpu.VMEM((4 * _B_PER_W,), jnp.int32),
        pltpu.VMEM((N_SPACE, N_SPACE), jnp.float32),
        pltpu.VMEM((LANES,), jnp.float32),
        pltpu.VMEM((_B_PER_W,), jnp.float32),
        pltpu.SemaphoreType.DMA,
        pltpu.SemaphoreType.DMA,
    ],
)
def _sc_kernel(xt_hbm, xpt_hbm, alpha_hbm, beta_hbm, out_hbm,
               xiv, av, bv, ov, sem_in, sem_ab):
    wid = lax.axis_index("s") * _NC + lax.axis_index("c")
    base = wid * _B_PER_W
    W = _B_PER_W

    c4 = pltpu.async_copy(alpha_hbm, av, sem_ab)
    c5 = pltpu.async_copy(beta_hbm, bv.at[pl.ds(0, 1)], sem_ab)
    c0 = pltpu.async_copy(xt_hbm.at[0, pl.ds(base, W)], xiv.at[pl.ds(0, W)], sem_in)
    c1 = pltpu.async_copy(xt_hbm.at[1, pl.ds(base, W)], xiv.at[pl.ds(W, W)], sem_in)
    c2 = pltpu.async_copy(xpt_hbm.at[0, pl.ds(base, W)], xiv.at[pl.ds(2 * W, W)], sem_in)
    c3 = pltpu.async_copy(xpt_hbm.at[1, pl.ds(base, W)], xiv.at[pl.ds(3 * W, W)], sem_in)
    c0.wait()
    c1.wait()
    c2.wait()
    c3.wait()
    c4.wait()
    c5.wait()

    beta = bv[...][0]

    @pl.loop(0, _VREGS, unroll=4)
    def _compute(j):
        o = j * LANES
        x0 = xiv[pl.ds(o, LANES)]
        x1 = xiv[pl.ds(W + o, LANES)]
        xp0 = xiv[pl.ds(2 * W + o, LANES)]
        xp1 = xiv[pl.ds(3 * W + o, LANES)]
        al = plsc.load_gather(av, [xp1, x1])
        dt = jnp.abs(x0 - xp0).astype(jnp.float32)
        ov[pl.ds(o, LANES)] = al * beta * jnp.exp(-beta * dt)

    pltpu.sync_copy(ov, out_hbm.at[pl.ds(base, W)])


def kernel(x, xp, alpha, beta):
    return _sc_kernel(x.T, xp.T, alpha, beta)


# speedup vs baseline: 1.0412x; 1.0412x over previous
import functools

import jax
import jax.numpy as jnp
from jax import lax
from jax.experimental import pallas as pl
from jax.experimental.pallas import tpu as pltpu
from jax.experimental.pallas import tpu_sc as plsc

BATCH = 16384
N_SPACE = 8
LANES = 16

_info = plsc.get_sparse_core_info()
_NC, _NS = 1, _info.num_subcores
_NW = _NC * _NS
_B_PER_W = BATCH // _NW
_VREGS = _B_PER_W // LANES

_mesh = plsc.VectorSubcoreMesh(
    core_axis_name="c", subcore_axis_name="s", num_cores=_NC)


@functools.partial(
    pl.kernel,
    mesh=_mesh,
    out_type=jax.ShapeDtypeStruct((BATCH,), jnp.float32),
    compiler_params=pltpu.CompilerParams(
        needs_layout_passes=False,
        disable_bounds_checks=True,
        disable_semaphore_checks=True,
        skip_device_barrier=True,
    ),
    scratch_types=[
        pltpu.VMEM((4 * _B_PER_W,), jnp.int32),
        pltpu.VMEM((N_SPACE, N_SPACE), jnp.float32),
        pltpu.VMEM((LANES,), jnp.float32),
        pltpu.VMEM((_B_PER_W,), jnp.float32),
        pltpu.SemaphoreType.DMA,
        pltpu.SemaphoreType.DMA,
    ],
)
def _sc_kernel(xt_hbm, xpt_hbm, alpha_hbm, beta_hbm, out_hbm,
               xiv, av, bv, ov, sem_in, sem_ab):
    wid = lax.axis_index("s") * _NC + lax.axis_index("c")
    base = wid * _B_PER_W
    W = _B_PER_W

    c4 = pltpu.async_copy(alpha_hbm, av, sem_ab)
    c5 = pltpu.async_copy(beta_hbm, bv.at[pl.ds(0, 1)], sem_ab)
    c0 = pltpu.async_copy(xt_hbm.at[0, pl.ds(base, W)], xiv.at[pl.ds(0, W)], sem_in)
    c1 = pltpu.async_copy(xt_hbm.at[1, pl.ds(base, W)], xiv.at[pl.ds(W, W)], sem_in)
    c2 = pltpu.async_copy(xpt_hbm.at[0, pl.ds(base, W)], xiv.at[pl.ds(2 * W, W)], sem_in)
    c3 = pltpu.async_copy(xpt_hbm.at[1, pl.ds(base, W)], xiv.at[pl.ds(3 * W, W)], sem_in)
    c0.wait()
    c1.wait()
    c2.wait()
    c3.wait()
    c4.wait()
    c5.wait()

    beta = bv[...][0]

    @pl.loop(0, _VREGS, unroll=4)
    def _compute(j):
        o = j * LANES
        x0 = xiv[pl.ds(o, LANES)]
        x1 = xiv[pl.ds(W + o, LANES)]
        xp0 = xiv[pl.ds(2 * W + o, LANES)]
        xp1 = xiv[pl.ds(3 * W + o, LANES)]
        al = plsc.load_gather(av, [xp1, x1])
        dt = jnp.abs(x0 - xp0).astype(jnp.float32)
        ov[pl.ds(o, LANES)] = al * beta * jnp.exp(-beta * dt)

    pltpu.sync_copy(ov, out_hbm.at[pl.ds(base, W)])


def kernel(x, xp, alpha, beta):
    return _sc_kernel(x.T, xp.T, alpha, beta)


# speedup vs baseline: 1.0598x; 1.0179x over previous
import functools

import jax
import jax.numpy as jnp
from jax import lax
from jax.experimental import pallas as pl
from jax.experimental.pallas import tpu as pltpu
from jax.experimental.pallas import tpu_sc as plsc

BATCH = 16384
N_SPACE = 8
LANES = 16

_info = plsc.get_sparse_core_info()
_NC, _NS = 1, _info.num_subcores
_NW = _NC * _NS
_B_PER_W = BATCH // _NW
_VREGS = _B_PER_W // LANES

_mesh = plsc.VectorSubcoreMesh(
    core_axis_name="c", subcore_axis_name="s", num_cores=_NC)


@functools.partial(
    pl.kernel,
    mesh=_mesh,
    out_type=jax.ShapeDtypeStruct((BATCH,), jnp.float32),
    compiler_params=pltpu.CompilerParams(
        needs_layout_passes=False,
        disable_bounds_checks=True,
        disable_semaphore_checks=True,
        skip_device_barrier=True,
    ),
    scratch_types=[
        pltpu.VMEM((4 * _B_PER_W,), jnp.int32),
        pltpu.VMEM((N_SPACE, N_SPACE), jnp.float32),
        pltpu.VMEM((LANES,), jnp.float32),
        pltpu.VMEM((_B_PER_W,), jnp.float32),
        pltpu.SemaphoreType.DMA,
        pltpu.SemaphoreType.DMA,
    ],
)
def _sc_kernel(xt_hbm, xpt_hbm, alpha_hbm, beta_hbm, out_hbm,
               xiv, av, bv, ov, sem_in, sem_ab):
    wid = lax.axis_index("s") * _NC + lax.axis_index("c")
    base = wid * _B_PER_W
    W = _B_PER_W

    c4 = pltpu.async_copy(alpha_hbm, av, sem_ab)
    c5 = pltpu.async_copy(beta_hbm, bv.at[pl.ds(0, 1)], sem_ab)
    c0 = pltpu.async_copy(xt_hbm.at[0, pl.ds(base, W)], xiv.at[pl.ds(0, W)], sem_in)
    c1 = pltpu.async_copy(xt_hbm.at[1, pl.ds(base, W)], xiv.at[pl.ds(W, W)], sem_in)
    c2 = pltpu.async_copy(xpt_hbm.at[0, pl.ds(base, W)], xiv.at[pl.ds(2 * W, W)], sem_in)
    c3 = pltpu.async_copy(xpt_hbm.at[1, pl.ds(base, W)], xiv.at[pl.ds(3 * W, W)], sem_in)
    c0.wait()
    c1.wait()
    c2.wait()
    c3.wait()
    c4.wait()
    c5.wait()

    beta = bv[...][0]

    @pl.loop(0, _VREGS)
    def _compute(j):
        o = j * LANES
        x0 = xiv[pl.ds(o, LANES)]
        x1 = xiv[pl.ds(W + o, LANES)]
        xp0 = xiv[pl.ds(2 * W + o, LANES)]
        xp1 = xiv[pl.ds(3 * W + o, LANES)]
        al = plsc.load_gather(av, [xp1, x1])
        dt = jnp.abs(x0 - xp0).astype(jnp.float32)
        ov[pl.ds(o, LANES)] = al * beta * jnp.exp(-beta * dt)

    pltpu.sync_copy(ov, out_hbm.at[pl.ds(base, W)])


def kernel(x, xp, alpha, beta):
    return _sc_kernel(x.T, xp.T, alpha, beta)
